# baseline (speedup 1.0000x reference)
import functools

import jax
import jax.numpy as jnp
from jax import lax
from jax.experimental import pallas as pl
from jax.experimental.pallas import tpu as pltpu
from jax.experimental.pallas import tpu_sc as plsc

BEV_SIZES = [(512, 512), (256, 256), (128, 128), (256, 256), (128, 128)]
V_COUNTS = [20000, 15000, 10000, 15000, 10000]
N_POINTS = 100000
NP = 100352
AVFE_DIM = 32
AVFEO_DIM = 64

BN = 2048
FMIN = float(jnp.finfo(jnp.float32).min)

CH = 6272
SC_G = NP // CH
VPAD2 = 20016
VPAD3 = 18816
POSL = 18816
ACC_ROWS = 40064
OFFS = [0, 0, 20008, 15008, 30016]

HW3 = 256 * 256
HW4 = 128 * 128


def _avfe_body(pT_ref, pm_ref, wptT_ref, wattT_ref, sf_ref):
    p = pT_ref[...]
    pf = jax.nn.relu(jnp.dot(wptT_ref[...], p, preferred_element_type=jnp.float32))
    pms = pm_ref[...]
    for k in range(3):
        pm = pms[4 * k:4 * k + 4]
        att = jnp.concatenate([p[:3] - pm[:3], p[3:4], pm], axis=0)
        af = jax.nn.relu(jnp.dot(wattT_ref[...], att, preferred_element_type=jnp.float32))
        sf_ref[32 * k:32 * k + 32, :] = pf * af


def _avfeo_body(pT_ref, sf012_ref, pvf012_ref, pm_ref,
                woTp_ref, woattT_ref, sf34_ref):
    p = pT_ref[...]
    finalT = jnp.concatenate([sf012_ref[...], pvf012_ref[...]], axis=0)
    pf = jax.nn.relu(jnp.dot(woTp_ref[...], finalT, preferred_element_type=jnp.float32))
    pms = pm_ref[...]
    for k in range(2):
        pm = pms[12 + 4 * k:16 + 4 * k]
        att = jnp.concatenate([p[:3] - pm[:3], p[3:4], pm], axis=0)
        af = jax.nn.relu(jnp.dot(woattT_ref[...], att, preferred_element_type=jnp.float32))
        sf34_ref[64 * k:64 * k + 64, :] = pf * af


def _whole(a):
    return pl.BlockSpec(a.shape, lambda i: (0,) * a.ndim)


def _true16():
    return jnp.broadcast_to(jnp.bool_(True), (16,))


def _rmw_max4(pairs):
    t16 = _true16()
    curs = [plsc.load_gather(acc, [idx], mask=t16) for acc, idx, _ in pairs]
    news = [jnp.maximum(c, v) for c, (_, _, v) in zip(curs, pairs)]
    for (acc, idx, _), new in zip(pairs, news):
        plsc.store_scatter(acc, [idx], new, mask=t16)
    chks = [plsc.load_gather(acc, [idx], mask=t16) for acc, idx, _ in pairs]
    pend = chks[0] < news[0]
    for c, n in zip(chks[1:], news[1:]):
        pend = pend | (c < n)

    @pl.when(jnp.any(pend))
    def _():
        def fix(_, __):
            for acc, idx, val in pairs:
                cur = plsc.load_gather(acc, [idx], mask=t16)
                plsc.store_scatter(acc, [idx], jnp.maximum(cur, val),
                                   mask=cur < val)
            return 0
        lax.fori_loop(0, 15, fix, 0)


def _sc_mesh():
    return plsc.VectorSubcoreMesh(core_axis_name="c", subcore_axis_name="s")


def _segsum_means(points8, pvsh2d, pvsh_flat, zrows):
    @functools.partial(
        pl.kernel,
        out_type=jax.ShapeDtypeStruct((20 * NP,), jnp.float32),
        mesh=_sc_mesh(),
        compiler_params=pltpu.CompilerParams(needs_layout_passes=False,
                                             use_tc_tiling_on_sc=False),
        scratch_types=[
            pltpu.VMEM((CH, 8), jnp.float32),
            pltpu.VMEM((56, 128), jnp.int32),
            pltpu.VMEM((CH,), jnp.int32),
            pltpu.VMEM((CH,), jnp.float32),
            pltpu.VMEM_SHARED((ACC_ROWS, 8), jnp.float32),
        ],
    )
    def run(p8_hbm, pv2_hbm, pvf_hbm, z_hbm, o_hbm,
            rowsb, idxb, idxg, outb, acc_sh):
        c = lax.axis_index("c")
        s = lax.axis_index("s")
        pltpu.sync_copy(z_hbm.at[pl.ds(0, 2504)],
                        acc_sh.at[pl.ds(s * 2504, 2504)])
        pltpu.sync_copy(p8_hbm.at[pl.ds(s * CH, CH)], rowsb)
        plsc.subcore_barrier()

        for k in range(3):
            scale = jnp.where(c == 0, 2 * k, 2 * k + 1)
            valid = (c == 0) | (k < 2)

            @pl.when(valid)
            def _():
                pltpu.sync_copy(pv2_hbm.at[pl.ds(scale * 896 + s * 56, 56)], idxb)

                def sadd(b, _):
                    pltpu.sync_copy(rowsb.at[pl.ds(b * 128, 128)],
                                    acc_sh.at[idxb.at[b]], add=True)
                    return 0
                lax.fori_loop(0, 49, sadd, 0)

        plsc.subcore_barrier()

        for k in range(3):
            scale = jnp.where(c == 0, 2 * k, 2 * k + 1)
            valid = (c == 0) | (k < 2)

            @pl.when(valid)
            def _():
                pltpu.sync_copy(pvf_hbm.at[pl.ds(scale * NP + s * CH, CH)], idxg)
                pltpu.sync_copy(acc_sh.at[idxg], rowsb)

                for c4 in range(4):
                    def div(j, _):
                        rows = j * 16 + lax.iota(jnp.int32, 16)
                        colv = jnp.full((16,), c4, jnp.int32)
                        colc = jnp.full((16,), 4, jnp.int32)
                        v = plsc.load_gather(rowsb, [rows, colv], mask=_true16())
                        n = plsc.load_gather(rowsb, [rows, colc], mask=_true16())
                        outb[pl.ds(j * 16, 16)] = v / jnp.maximum(n, 1.0)
                        return 0
                    lax.fori_loop(0, CH // 16, div, 0)
                    pltpu.sync_copy(
                        outb,
                        o_hbm.at[pl.ds((scale * 4 + c4) * NP + s * CH, CH)])

    return run(points8, pvsh2d, pvsh_flat, zrows).reshape(20, NP)


def _segmax_gather_012(sf012, pv012):
    @functools.partial(
        pl.kernel,
        out_type=jax.ShapeDtypeStruct((3 * AVFE_DIM * NP,), jnp.float32),
        mesh=_sc_mesh(),
        compiler_params=pltpu.CompilerParams(needs_layout_passes=False),
        scratch_types=[
            pltpu.VMEM((VPAD2,), jnp.float32),
            pltpu.VMEM((CH,), jnp.int32),
            pltpu.VMEM((CH,), jnp.float32),
            pltpu.VMEM((CH,), jnp.float32),
        ],
    )
    def run(sf_hbm, pv_hbm, o_hbm, acc, idxb, valb, outb):
        ch = lax.axis_index("s") * 2 + lax.axis_index("c")
        neg = jnp.full((16,), FMIN, jnp.float32)

        def scale_body(i, _):
            def initb(k, _):
                acc[pl.ds(k * 16, 16)] = neg
                return 0
            lax.fori_loop(0, VPAD2 // 16, initb, 0)

            def chunk_acc(g, _):
                pltpu.sync_copy(pv_hbm.at[pl.ds(i * NP + g * CH, CH)], idxb)
                pltpu.sync_copy(
                    sf_hbm.at[pl.ds((i * AVFE_DIM + ch) * NP + g * CH, CH)], valb)

                def step(j, _):
                    _rmw_max4([(acc, idxb[pl.ds(j * 128 + 16 * u, 16)],
                                valb[pl.ds(j * 128 + 16 * u, 16)])
                               for u in range(8)])
                    return 0
                lax.fori_loop(0, CH // 128, step, 0)
                return 0
            lax.fori_loop(0, SC_G, chunk_acc, 0)

            def chunk_g(g, _):
                pltpu.sync_copy(pv_hbm.at[pl.ds(i * NP + g * CH, CH)], idxb)

                def gstep(j, _):
                    for u in range(8):
                        idx = idxb[pl.ds(j * 128 + 16 * u, 16)]
                        outb[pl.ds(j * 128 + 16 * u, 16)] = plsc.load_gather(
                            acc, [idx], mask=_true16())
                    return 0
                lax.fori_loop(0, CH // 128, gstep, 0)
                pltpu.sync_copy(
                    outb,
                    o_hbm.at[pl.ds((i * AVFE_DIM + ch) * NP + g * CH, CH)])
                return 0
            lax.fori_loop(0, SC_G, chunk_g, 0)
            return 0
        lax.fori_loop(0, 3, scale_body, 0)

    return run(sf012, pv012).reshape(3 * AVFE_DIM, NP)


def _segmax_dense_34(sf34, pv34, pos_all):
    @functools.partial(
        pl.kernel,
        out_type=jax.ShapeDtypeStruct((AVFEO_DIM * (HW3 + HW4),), jnp.float32),
        mesh=_sc_mesh(),
        compiler_params=pltpu.CompilerParams(needs_layout_passes=False),
        scratch_types=[
            pltpu.VMEM((VPAD3,), jnp.float32),
            pltpu.VMEM((VPAD3,), jnp.float32),
            pltpu.VMEM((CH,), jnp.int32),
            pltpu.VMEM((CH,), jnp.float32),
            pltpu.VMEM((CH,), jnp.float32),
            pltpu.VMEM((HW3,), jnp.float32),
        ],
    )
    def run(sf_hbm, pv_hbm, pos_hbm, o_hbm,
            acc0, acc1, idxb, valb0, valb1, rowb):
        wid = lax.axis_index("s") * 2 + lax.axis_index("c")
        neg = jnp.full((16,), FMIN, jnp.float32)
        zero = jnp.zeros((16,), jnp.float32)

        def scale_body(i, _):
            V = jnp.where(i == 0, V_COUNTS[3], V_COUNTS[4])
            hw = jnp.where(i == 0, HW3, HW4)
            vg = jnp.where(i == 0, 3, 2)
            nrow = jnp.where(i == 0, 4, 1)
            obase = jnp.where(i == 0, 0, AVFEO_DIM * HW3)

            def initb(k, _):
                acc0[pl.ds(k * 16, 16)] = neg
                acc1[pl.ds(k * 16, 16)] = neg
                return 0
            lax.fori_loop(0, VPAD3 // 16, initb, 0)

            def chunk_acc(g, _):
                pltpu.sync_copy(pv_hbm.at[pl.ds(i * NP + g * CH, CH)], idxb)
                pltpu.sync_copy(
                    sf_hbm.at[pl.ds((i * AVFEO_DIM + 2 * wid) * NP + g * CH, CH)],
                    valb0)
                pltpu.sync_copy(
                    sf_hbm.at[pl.ds((i * AVFEO_DIM + 2 * wid + 1) * NP + g * CH, CH)],
                    valb1)

                def step(j, _):
                    idxs = [idxb[pl.ds(j * 128 + 16 * u, 16)] for u in range(8)]
                    _rmw_max4(
                        [(acc0, idxs[u], valb0[pl.ds(j * 128 + 16 * u, 16)])
                         for u in range(8)]
                        + [(acc1, idxs[u], valb1[pl.ds(j * 128 + 16 * u, 16)])
                           for u in range(8)])
                    return 0
                lax.fori_loop(0, CH // 128, step, 0)
                return 0
            lax.fori_loop(0, SC_G, chunk_acc, 0)

            for kc, acck in ((0, acc0), (1, acc1)):
                def memset(m, _):
                    rowb[pl.ds(m * 16, 16)] = zero
                    return 0
                lax.fori_loop(0, hw // 16, memset, 0)

                def scat(vgi, _):
                    pltpu.sync_copy(
                        pos_hbm.at[pl.ds(i * POSL + vgi * CH, CH)], idxb)

                    def sstep(j, _):
                        lane = vgi * CH + j * 16 + lax.iota(jnp.int32, 16)
                        pos = idxb[pl.ds(j * 16, 16)]
                        v = acck[pl.ds(vgi * CH + j * 16, 16)]
                        plsc.store_scatter(rowb, [pos], v, mask=lane < V)
                        return 0
                    lax.fori_loop(0, CH // 16, sstep, 0)
                    return 0
                lax.fori_loop(0, vg, scat, 0)

                def wout(r, _):
                    pltpu.sync_copy(
                        rowb.at[pl.ds(r * HW4, HW4)],
                        o_hbm.at[pl.ds(obase + (2 * wid + kc) * hw + r * HW4, HW4)])
                    return 0
                lax.fori_loop(0, nrow, wout, 0)
            return 0
        lax.fori_loop(0, 2, scale_body, 0)

    return run(sf34, pv34, pos_all)


def kernel(points, pv0, pv1, pv2, pv3, pv4, vf0, vf1, vf2, vf3, vf4,
           W_avfe_pt, W_avfe_att, W_avfeo_pt, W_avfeo_att, batch_size):
    pvs = [pv0, pv1, pv2, pv3, pv4]
    vfs = [vf0, vf1, vf2, vf3, vf4]
    N = points.shape[0]
    pad = NP - N
    grid = (NP // BN,)

    points_pad = jnp.concatenate([points, jnp.zeros((pad, 4), jnp.float32)], axis=0)
    pT = points_pad.T
    ones_col = jnp.concatenate([jnp.ones((N, 1), jnp.float32),
                                jnp.zeros((pad, 1), jnp.float32)], axis=0)
    points8 = jnp.concatenate([points_pad, ones_col,
                               jnp.zeros((NP, 3), jnp.float32)], axis=1)
    pvp = [jnp.concatenate([pvs[i], jnp.full((pad,), V_COUNTS[i], jnp.int32)])
           for i in range(5)]
    pvsh_flat = jnp.concatenate([pvp[i] + OFFS[i] for i in range(5)])
    pvsh2d = jnp.pad(pvsh_flat.reshape(5, 16, 49, 128),
                     ((0, 0), (0, 0), (0, 7), (0, 0))).reshape(5 * 896, 128)
    zrows = jnp.zeros((2504, 8), jnp.float32)
    pos = []
    for i in (3, 4):
        H, W = BEV_SIZES[i]
        p = vfs[i][:, 1] * W + vfs[i][:, 2]
        pos.append(jnp.concatenate([p, jnp.zeros((POSL - V_COUNTS[i],), jnp.int32)]))
    pos_all = jnp.concatenate(pos)

    wptT = W_avfe_pt.T
    wattT = W_avfe_att.T
    woattT = W_avfeo_att.T
    perm = jnp.array([0, 2, 4, 1, 3, 5], jnp.int32)
    woTp = W_avfeo_pt.reshape(6, AVFE_DIM, AVFEO_DIM)[perm].reshape(192, AVFEO_DIM).T

    pmT = _segsum_means(points8, pvsh2d, pvsh_flat, zrows)

    sf012 = pl.pallas_call(
        _avfe_body,
        grid=grid,
        in_specs=[pl.BlockSpec((4, BN), lambda i: (0, i)),
                  pl.BlockSpec((20, BN), lambda i: (0, i)),
                  _whole(wptT), _whole(wattT)],
        out_specs=pl.BlockSpec((3 * AVFE_DIM, BN), lambda i: (0, i)),
        out_shape=jax.ShapeDtypeStruct((3 * AVFE_DIM, NP), jnp.float32),
    )(pT, pmT, wptT, wattT)

    pv012 = jnp.concatenate(pvp[:3])
    pvf012 = _segmax_gather_012(sf012.reshape(-1), pv012)

    sf34 = pl.pallas_call(
        _avfeo_body,
        grid=grid,
        in_specs=[pl.BlockSpec((4, BN), lambda i: (0, i)),
                  pl.BlockSpec((3 * AVFE_DIM, BN), lambda i: (0, i)),
                  pl.BlockSpec((3 * AVFE_DIM, BN), lambda i: (0, i)),
                  pl.BlockSpec((20, BN), lambda i: (0, i)),
                  _whole(woTp), _whole(woattT)],
        out_specs=pl.BlockSpec((2 * AVFEO_DIM, BN), lambda i: (0, i)),
        out_shape=jax.ShapeDtypeStruct((2 * AVFEO_DIM, NP), jnp.float32),
    )(pT, sf012, pvf012, pmT, woTp, woattT)

    pv34 = jnp.concatenate(pvp[3:])
    dense = _segmax_dense_34(sf34.reshape(-1), pv34, pos_all)
    out3 = dense[:AVFEO_DIM * HW3].reshape(1, AVFEO_DIM, 256, 256)
    out4 = dense[AVFEO_DIM * HW3:].reshape(1, AVFEO_DIM, 128, 128)
    return (out3, out4)


# speedup vs baseline: 1.3268x; 1.3268x over previous
import functools

import jax
import jax.numpy as jnp
from jax import lax
from jax.experimental import pallas as pl
from jax.experimental.pallas import tpu as pltpu
from jax.experimental.pallas import tpu_sc as plsc

BEV_SIZES = [(512, 512), (256, 256), (128, 128), (256, 256), (128, 128)]
V_COUNTS = [20000, 15000, 10000, 15000, 10000]
N_POINTS = 100000
NP = 100352
AVFE_DIM = 32
AVFEO_DIM = 64

BN = 2048
FMIN = float(jnp.finfo(jnp.float32).min)

CH = 6272
SC_G = NP // CH
VPAD2 = 20016
VPAD3 = 18816
POSL = 18816
ACC_ROWS = 40064
OFFS = [0, 0, 20008, 15008, 30016]

HW3 = 256 * 256
HW4 = 128 * 128


def _avfe_body(pT_ref, pm_ref, wptT_ref, wattT_ref, sf_ref):
    p = pT_ref[...]
    pf = jax.nn.relu(jnp.dot(wptT_ref[...], p, preferred_element_type=jnp.float32))
    pms = pm_ref[...]
    for k in range(3):
        pm = pms[4 * k:4 * k + 4]
        att = jnp.concatenate([p[:3] - pm[:3], p[3:4], pm], axis=0)
        af = jax.nn.relu(jnp.dot(wattT_ref[...], att, preferred_element_type=jnp.float32))
        sf_ref[32 * k:32 * k + 32, :] = pf * af


def _avfeo_body(pT_ref, sf012_ref, pvf012_ref, pm_ref,
                woTp_ref, woattT_ref, sf34_ref):
    p = pT_ref[...]
    finalT = jnp.concatenate([sf012_ref[...], pvf012_ref[...]], axis=0)
    pf = jax.nn.relu(jnp.dot(woTp_ref[...], finalT, preferred_element_type=jnp.float32))
    pms = pm_ref[...]
    for k in range(2):
        pm = pms[12 + 4 * k:16 + 4 * k]
        att = jnp.concatenate([p[:3] - pm[:3], p[3:4], pm], axis=0)
        af = jax.nn.relu(jnp.dot(woattT_ref[...], att, preferred_element_type=jnp.float32))
        sf34_ref[64 * k:64 * k + 64, :] = pf * af


def _whole(a):
    return pl.BlockSpec(a.shape, lambda i: (0,) * a.ndim)


def _true16():
    return jnp.broadcast_to(jnp.bool_(True), (16,))


def _rmw_max4(pairs):
    t16 = _true16()
    curs = [plsc.load_gather(acc, [idx], mask=t16) for acc, idx, _ in pairs]
    news = [jnp.maximum(c, v) for c, (_, _, v) in zip(curs, pairs)]
    for (acc, idx, _), new in zip(pairs, news):
        plsc.store_scatter(acc, [idx], new, mask=t16)
    chks = [plsc.load_gather(acc, [idx], mask=t16) for acc, idx, _ in pairs]
    pend = chks[0] < news[0]
    for c, n in zip(chks[1:], news[1:]):
        pend = pend | (c < n)

    @pl.when(jnp.any(pend))
    def _():
        def fix(_, __):
            for acc, idx, val in pairs:
                cur = plsc.load_gather(acc, [idx], mask=t16)
                plsc.store_scatter(acc, [idx], jnp.maximum(cur, val),
                                   mask=cur < val)
            return 0
        lax.fori_loop(0, 15, fix, 0)


def _sc_mesh():
    return plsc.VectorSubcoreMesh(core_axis_name="c", subcore_axis_name="s")


def _segsum_means(points8, pvsh2d, pvsh_flat, zrows):
    @functools.partial(
        pl.kernel,
        out_type=jax.ShapeDtypeStruct((20 * NP,), jnp.float32),
        mesh=_sc_mesh(),
        compiler_params=pltpu.CompilerParams(needs_layout_passes=False,
                                             use_tc_tiling_on_sc=False),
        scratch_types=[
            pltpu.VMEM((CH, 8), jnp.float32),
            pltpu.VMEM((56, 128), jnp.int32),
            pltpu.VMEM((CH,), jnp.int32),
            pltpu.VMEM((CH,), jnp.float32),
            pltpu.VMEM_SHARED((ACC_ROWS, 8), jnp.float32),
        ],
    )
    def run(p8_hbm, pv2_hbm, pvf_hbm, z_hbm, o_hbm,
            rowsb, idxb, idxg, outb, acc_sh):
        c = lax.axis_index("c")
        s = lax.axis_index("s")
        pltpu.sync_copy(z_hbm.at[pl.ds(0, 2504)],
                        acc_sh.at[pl.ds(s * 2504, 2504)])
        pltpu.sync_copy(p8_hbm.at[pl.ds(s * CH, CH)], rowsb)
        plsc.subcore_barrier()

        for k in range(3):
            scale = jnp.where(c == 0, 2 * k, 2 * k + 1)
            valid = (c == 0) | (k < 2)

            @pl.when(valid)
            def _():
                pltpu.sync_copy(pv2_hbm.at[pl.ds(scale * 896 + s * 56, 56)], idxb)

                def sadd(b, _):
                    pltpu.sync_copy(rowsb.at[pl.ds(b * 128, 128)],
                                    acc_sh.at[idxb.at[b]], add=True)
                    return 0
                lax.fori_loop(0, 49, sadd, 0)

        plsc.subcore_barrier()

        for k in range(3):
            scale = jnp.where(c == 0, 2 * k, 2 * k + 1)
            valid = (c == 0) | (k < 2)

            @pl.when(valid)
            def _():
                pltpu.sync_copy(pvf_hbm.at[pl.ds(scale * NP + s * CH, CH)], idxg)
                pltpu.sync_copy(acc_sh.at[idxg], rowsb)

                for c4 in range(4):
                    def div(j, _):
                        rows = j * 16 + lax.iota(jnp.int32, 16)
                        colv = jnp.full((16,), c4, jnp.int32)
                        colc = jnp.full((16,), 4, jnp.int32)
                        v = plsc.load_gather(rowsb, [rows, colv], mask=_true16())
                        n = plsc.load_gather(rowsb, [rows, colc], mask=_true16())
                        outb[pl.ds(j * 16, 16)] = v / jnp.maximum(n, 1.0)
                        return 0
                    lax.fori_loop(0, CH // 16, div, 0)
                    pltpu.sync_copy(
                        outb,
                        o_hbm.at[pl.ds((scale * 4 + c4) * NP + s * CH, CH)])

    return run(points8, pvsh2d, pvsh_flat, zrows).reshape(20, NP)


def _segmax_gather_012(sf012, pv012):
    @functools.partial(
        pl.kernel,
        out_type=jax.ShapeDtypeStruct((3 * AVFE_DIM * NP,), jnp.float32),
        mesh=_sc_mesh(),
        compiler_params=pltpu.CompilerParams(needs_layout_passes=False),
        scratch_types=[
            pltpu.VMEM((VPAD2,), jnp.float32),
            pltpu.VMEM((CH,), jnp.int32),
            pltpu.VMEM((CH,), jnp.float32),
            pltpu.VMEM((CH,), jnp.float32),
        ],
    )
    def run(sf_hbm, pv_hbm, o_hbm, acc, idxb, valb, outb):
        ch = lax.axis_index("s") * 2 + lax.axis_index("c")
        neg = jnp.full((16,), FMIN, jnp.float32)

        def scale_body(i, _):
            def initb(k, _):
                acc[pl.ds(k * 16, 16)] = neg
                return 0
            lax.fori_loop(0, VPAD2 // 16, initb, 0)

            def chunk_acc(g, _):
                pltpu.sync_copy(pv_hbm.at[pl.ds(i * NP + g * CH, CH)], idxb)
                pltpu.sync_copy(
                    sf_hbm.at[pl.ds((i * AVFE_DIM + ch) * NP + g * CH, CH)], valb)

                def step(j, _):
                    _rmw_max4([(acc, idxb[pl.ds(j * 64 + 16 * u, 16)],
                                valb[pl.ds(j * 64 + 16 * u, 16)])
                               for u in range(4)])
                    return 0
                lax.fori_loop(0, CH // 64, step, 0)
                return 0
            lax.fori_loop(0, SC_G, chunk_acc, 0)

            def chunk_g(g, _):
                pltpu.sync_copy(pv_hbm.at[pl.ds(i * NP + g * CH, CH)], idxb)

                def gstep(j, _):
                    for u in range(4):
                        idx = idxb[pl.ds(j * 64 + 16 * u, 16)]
                        outb[pl.ds(j * 64 + 16 * u, 16)] = plsc.load_gather(
                            acc, [idx], mask=_true16())
                    return 0
                lax.fori_loop(0, CH // 64, gstep, 0)
                pltpu.sync_copy(
                    outb,
                    o_hbm.at[pl.ds((i * AVFE_DIM + ch) * NP + g * CH, CH)])
                return 0
            lax.fori_loop(0, SC_G, chunk_g, 0)
            return 0
        lax.fori_loop(0, 3, scale_body, 0)

    return run(sf012, pv012).reshape(3 * AVFE_DIM, NP)


def _segmax_dense_34(sf34, pv34, pos_all):
    @functools.partial(
        pl.kernel,
        out_type=[jax.ShapeDtypeStruct((AVFEO_DIM * HW3,), jnp.float32),
                  jax.ShapeDtypeStruct((AVFEO_DIM * HW4,), jnp.float32)],
        mesh=_sc_mesh(),
        compiler_params=pltpu.CompilerParams(needs_layout_passes=False),
        scratch_types=[
            pltpu.VMEM((VPAD3,), jnp.float32),
            pltpu.VMEM((VPAD3,), jnp.float32),
            pltpu.VMEM((CH,), jnp.int32),
            pltpu.VMEM((CH,), jnp.float32),
            pltpu.VMEM((CH,), jnp.float32),
            pltpu.VMEM((HW3,), jnp.float32),
        ],
    )
    def run(sf_hbm, pv_hbm, pos_hbm, o3_hbm, o4_hbm,
            acc0, acc1, idxb, valb0, valb1, rowb):
        wid = lax.axis_index("s") * 2 + lax.axis_index("c")
        neg = jnp.full((16,), FMIN, jnp.float32)
        zero = jnp.zeros((16,), jnp.float32)

        for i, (o_hbm, V, hw) in enumerate(((o3_hbm, V_COUNTS[3], HW3),
                                            (o4_hbm, V_COUNTS[4], HW4))):
            vg = -(-V // CH)

            def initb(k, _):
                acc0[pl.ds(k * 16, 16)] = neg
                acc1[pl.ds(k * 16, 16)] = neg
                return 0
            lax.fori_loop(0, VPAD3 // 16, initb, 0)

            def chunk_acc(g, _):
                pltpu.sync_copy(pv_hbm.at[pl.ds(i * NP + g * CH, CH)], idxb)
                pltpu.sync_copy(
                    sf_hbm.at[pl.ds((i * AVFEO_DIM + 2 * wid) * NP + g * CH, CH)],
                    valb0)
                pltpu.sync_copy(
                    sf_hbm.at[pl.ds((i * AVFEO_DIM + 2 * wid + 1) * NP + g * CH, CH)],
                    valb1)

                def step(j, _):
                    idxs = [idxb[pl.ds(j * 64 + 16 * u, 16)] for u in range(4)]
                    _rmw_max4(
                        [(acc0, idxs[u], valb0[pl.ds(j * 64 + 16 * u, 16)])
                         for u in range(4)]
                        + [(acc1, idxs[u], valb1[pl.ds(j * 64 + 16 * u, 16)])
                           for u in range(4)])
                    return 0
                lax.fori_loop(0, CH // 64, step, 0)
                return 0
            lax.fori_loop(0, SC_G, chunk_acc, 0)

            for kc, acck in ((0, acc0), (1, acc1)):
                def memset(m, _):
                    rowb[pl.ds(m * 16, 16)] = zero
                    return 0
                lax.fori_loop(0, hw // 16, memset, 0)

                def scat(vgi, _):
                    pltpu.sync_copy(
                        pos_hbm.at[pl.ds(i * POSL + vgi * CH, CH)], idxb)

                    def sstep(j, _):
                        base = vgi * CH + j * 64
                        for u in range(4):
                            lane = base + 16 * u + lax.iota(jnp.int32, 16)
                            pos = idxb[pl.ds(j * 64 + 16 * u, 16)]
                            v = acck[pl.ds(base + 16 * u, 16)]
                            plsc.store_scatter(rowb, [pos], v, mask=lane < V)
                        return 0
                    lax.fori_loop(0, CH // 64, sstep, 0)
                    return 0
                lax.fori_loop(0, vg, scat, 0)

                def wout(r, _):
                    pltpu.sync_copy(
                        rowb.at[pl.ds(r * HW4, HW4)],
                        o_hbm.at[pl.ds((2 * wid + kc) * hw + r * HW4, HW4)])
                    return 0
                lax.fori_loop(0, hw // HW4, wout, 0)

    return run(sf34, pv34, pos_all)


def kernel(points, pv0, pv1, pv2, pv3, pv4, vf0, vf1, vf2, vf3, vf4,
           W_avfe_pt, W_avfe_att, W_avfeo_pt, W_avfeo_att, batch_size):
    pvs = [pv0, pv1, pv2, pv3, pv4]
    vfs = [vf0, vf1, vf2, vf3, vf4]
    N = points.shape[0]
    pad = NP - N
    grid = (NP // BN,)

    points_pad = jnp.concatenate([points, jnp.zeros((pad, 4), jnp.float32)], axis=0)
    pT = points_pad.T
    ones_col = jnp.concatenate([jnp.ones((N, 1), jnp.float32),
                                jnp.zeros((pad, 1), jnp.float32)], axis=0)
    points8 = jnp.concatenate([points_pad, ones_col,
                               jnp.zeros((NP, 3), jnp.float32)], axis=1)
    pvp = [jnp.concatenate([pvs[i], jnp.full((pad,), V_COUNTS[i], jnp.int32)])
           for i in range(5)]
    pvsh_flat = jnp.concatenate([pvp[i] + OFFS[i] for i in range(5)])
    pvsh2d = jnp.pad(pvsh_flat.reshape(5, 16, 49, 128),
                     ((0, 0), (0, 0), (0, 7), (0, 0))).reshape(5 * 896, 128)
    zrows = jnp.zeros((2504, 8), jnp.float32)
    pos = []
    for i in (3, 4):
        H, W = BEV_SIZES[i]
        p = vfs[i][:, 1] * W + vfs[i][:, 2]
        pos.append(jnp.concatenate([p, jnp.zeros((POSL - V_COUNTS[i],), jnp.int32)]))
    pos_all = jnp.concatenate(pos)

    wptT = W_avfe_pt.T
    wattT = W_avfe_att.T
    woattT = W_avfeo_att.T
    perm = jnp.array([0, 2, 4, 1, 3, 5], jnp.int32)
    woTp = W_avfeo_pt.reshape(6, AVFE_DIM, AVFEO_DIM)[perm].reshape(192, AVFEO_DIM).T

    pmT = _segsum_means(points8, pvsh2d, pvsh_flat, zrows)

    sf012 = pl.pallas_call(
        _avfe_body,
        grid=grid,
        in_specs=[pl.BlockSpec((4, BN), lambda i: (0, i)),
                  pl.BlockSpec((20, BN), lambda i: (0, i)),
                  _whole(wptT), _whole(wattT)],
        out_specs=pl.BlockSpec((3 * AVFE_DIM, BN), lambda i: (0, i)),
        out_shape=jax.ShapeDtypeStruct((3 * AVFE_DIM, NP), jnp.float32),
    )(pT, pmT, wptT, wattT)

    pv012 = jnp.concatenate(pvp[:3])
    pvf012 = _segmax_gather_012(sf012.reshape(-1), pv012)

    sf34 = pl.pallas_call(
        _avfeo_body,
        grid=grid,
        in_specs=[pl.BlockSpec((4, BN), lambda i: (0, i)),
                  pl.BlockSpec((3 * AVFE_DIM, BN), lambda i: (0, i)),
                  pl.BlockSpec((3 * AVFE_DIM, BN), lambda i: (0, i)),
                  pl.BlockSpec((20, BN), lambda i: (0, i)),
                  _whole(woTp), _whole(woattT)],
        out_specs=pl.BlockSpec((2 * AVFEO_DIM, BN), lambda i: (0, i)),
        out_shape=jax.ShapeDtypeStruct((2 * AVFEO_DIM, NP), jnp.float32),
    )(pT, sf012, pvf012, pmT, woTp, woattT)

    pv34 = jnp.concatenate(pvp[3:])
    d3, d4 = _segmax_dense_34(sf34.reshape(-1), pv34, pos_all)
    return (d3.reshape(1, AVFEO_DIM, 256, 256),
            d4.reshape(1, AVFEO_DIM, 128, 128))
